# Initial kernel scaffold; baseline (speedup 1.0000x reference)
import jax
import jax.numpy as jnp
from jax.experimental import pallas as pl


def kernel(indices, hashed_weight, random_numbers):
    raise NotImplementedError("write your pallas kernel here")


# speedup vs baseline: 162.4784x; 162.4784x over previous
import functools

import jax
import jax.numpy as jnp
from jax import lax
from jax.experimental import pallas as pl
from jax.experimental.pallas import tpu as pltpu
from jax.experimental.pallas import tpu_sc as plsc

EMB = 64
WEIGHT_SIZE = 1000001
N_ROWS = 4096 * 50
NC, NS, LANES = 2, 16, 16
NW = NC * NS
ROWS_PER_TILE = N_ROWS // NW
CHUNK_ROWS = 400
N_CHUNKS = ROWS_PER_TILE // CHUNK_ROWS
CHUNK_ELEMS = CHUNK_ROWS * EMB
GROUPS = CHUNK_ROWS // LANES


_MESH = plsc.VectorSubcoreMesh(core_axis_name="c", subcore_axis_name="s")


@functools.partial(
    pl.kernel,
    mesh=_MESH,
    out_type=jax.ShapeDtypeStruct((N_ROWS * EMB,), jnp.float32),
    scratch_types=[
        pltpu.VMEM((4, 16), jnp.int32),
        pltpu.VMEM((1024,), jnp.int32),
        pltpu.VMEM((1024,), jnp.int32),
        pltpu.VMEM((32768,), jnp.int32),
        pltpu.VMEM((CHUNK_ROWS,), jnp.int32),
        pltpu.VMEM((CHUNK_ELEMS,), jnp.int32),
        pltpu.VMEM((CHUNK_ELEMS,), jnp.float32),
        pltpu.SemaphoreType.DMA,
    ],
    compiler_params=pltpu.CompilerParams(needs_layout_passes=False),
)
def _hash_gather(idx_hbm, w_hbm, t0_hbm, t1_hbm, t2_hbm, par_hbm, out_hbm,
                 par_v, t0_v, t1_v, t2_v, idx_v, hidx_v, gath_v, sem):
    wid = lax.axis_index("s") * jnp.int32(NC) + lax.axis_index("c")
    pltpu.sync_copy(t0_hbm, t0_v)
    pltpu.sync_copy(t1_hbm, t1_v)
    pltpu.sync_copy(t2_hbm, t2_v)
    pltpu.sync_copy(par_hbm, par_v)
    pv = par_v[0, :]
    bv = par_v[1, :]
    pmav = par_v[2, :]
    wv = par_v[3, :]
    lanes = lax.iota(jnp.int32, 16)
    tile_row0 = wid * jnp.int32(ROWS_PER_TILE)

    def chunk_body(c, carry):
        row0 = tile_row0 + c * jnp.int32(CHUNK_ROWS)
        pltpu.sync_copy(idx_hbm.at[pl.ds(row0, CHUNK_ROWS)], idx_v)

        def group_body(g, inner):
            rix = g * jnp.int32(LANES) + lanes
            idxv = plsc.load_gather(idx_v, [rix])
            ih = lax.shift_right_logical(idxv, jnp.int32(10))
            il = lax.bitwise_and(idxv, jnp.int32(1023))
            t1v = plsc.load_gather(t1_v, [ih])
            t0v = plsc.load_gather(t0_v, [il])
            base = t1v - pv + t0v
            base = jnp.where(base < 0, base + pv, base)
            t = base - pv + bv
            t = jnp.where(t < 0, t + pv, t)
            pos = rix * jnp.int32(EMB)
            for d in range(EMB):
                hh = lax.shift_right_logical(t, jnp.int32(16))
                hl = lax.bitwise_and(t, jnp.int32(0xFFFF))
                t2v = plsc.load_gather(t2_v, [hh])
                r = t2v + hl
                r = jnp.where(r >= wv, r - wv, r)
                plsc.store_scatter(hidx_v, [pos + jnp.int32(d)], r)
                if d != EMB - 1:
                    t = t - pmav
                    t = jnp.where(t < 0, t + pv, t)
            return inner

        lax.fori_loop(jnp.int32(0), jnp.int32(GROUPS), group_body, jnp.int32(0))
        pltpu.async_copy(w_hbm.at[hidx_v], gath_v, sem).wait()
        pltpu.sync_copy(gath_v, out_hbm.at[pl.ds(row0 * jnp.int32(EMB), CHUNK_ELEMS)])
        return carry

    lax.fori_loop(jnp.int32(0), jnp.int32(N_CHUNKS), chunk_body, jnp.int32(0))


def kernel(indices, hashed_weight, random_numbers):
    i_shape = indices.shape
    rn = random_numbers.astype(jnp.int64)
    p, a, b = rn[0], rn[1], rn[2]
    a64 = (a * EMB) % p
    a64k = (a64 * 1024) % p
    j = jnp.arange(1024, dtype=jnp.int64)
    t1 = ((a64k * j) % p).astype(jnp.int32)
    t0 = ((a64 * j) % p).astype(jnp.int32)
    hh = jnp.arange(32768, dtype=jnp.int64)
    t2 = ((hh * 65536) % WEIGHT_SIZE).astype(jnp.int32)
    par = jnp.stack([p, b, p - a, jnp.int64(WEIGHT_SIZE)]).astype(jnp.int32)
    par = jnp.broadcast_to(par[:, None], (4, 16))
    idx_flat = indices.reshape(-1).astype(jnp.int32)
    out = _hash_gather(idx_flat, hashed_weight, t0, t1, t2, par)
    return out.reshape(*i_shape, EMB)


# speedup vs baseline: 233.3759x; 1.4364x over previous
import functools

import jax
import jax.numpy as jnp
from jax import lax
from jax.experimental import pallas as pl
from jax.experimental.pallas import tpu as pltpu
from jax.experimental.pallas import tpu_sc as plsc

EMB = 64
WEIGHT_SIZE = 1000001
N_ROWS = 4096 * 50
NC, NS, LANES = 2, 16, 16
NW = NC * NS
ROWS_PER_TILE = N_ROWS // NW
CHUNK_ROWS = 320
N_CHUNKS = ROWS_PER_TILE // CHUNK_ROWS
CHUNK_ELEMS = CHUNK_ROWS * EMB
GROUPS = CHUNK_ROWS // LANES


_MESH = plsc.VectorSubcoreMesh(core_axis_name="c", subcore_axis_name="s")


@functools.partial(
    pl.kernel,
    mesh=_MESH,
    out_type=jax.ShapeDtypeStruct((N_ROWS * EMB,), jnp.float32),
    scratch_types=[
        pltpu.VMEM((4, 16), jnp.int32),
        pltpu.VMEM((1024,), jnp.int32),
        pltpu.VMEM((1024,), jnp.int32),
        pltpu.VMEM((32768,), jnp.int32),
        pltpu.VMEM((ROWS_PER_TILE,), jnp.int32),
        pltpu.VMEM((CHUNK_ELEMS,), jnp.int32),
        pltpu.VMEM((CHUNK_ELEMS,), jnp.int32),
        pltpu.VMEM((CHUNK_ELEMS,), jnp.float32),
        pltpu.VMEM((CHUNK_ELEMS,), jnp.float32),
        pltpu.SemaphoreType.DMA,
        pltpu.SemaphoreType.DMA,
    ],
    compiler_params=pltpu.CompilerParams(needs_layout_passes=False),
)
def _hash_gather(idx_hbm, w_hbm, t0_hbm, t1_hbm, t2_hbm, par_hbm, out_hbm,
                 par_v, t0_v, t1_v, t2_v, idx_v, hidx_v0, hidx_v1,
                 gath_v0, gath_v1, sem0, sem1):
    wid = lax.axis_index("s") * jnp.int32(NC) + lax.axis_index("c")
    tile_row0 = wid * jnp.int32(ROWS_PER_TILE)
    pltpu.sync_copy(t0_hbm, t0_v)
    pltpu.sync_copy(t1_hbm, t1_v)
    pltpu.sync_copy(t2_hbm, t2_v)
    pltpu.sync_copy(par_hbm, par_v)
    pltpu.sync_copy(idx_hbm.at[pl.ds(tile_row0, ROWS_PER_TILE)], idx_v)
    pv = par_v[0, :]
    bv = par_v[1, :]
    pmav = par_v[2, :]
    wv = par_v[3, :]
    lanes = lax.iota(jnp.int32, 16)
    hidx_b = (hidx_v0, hidx_v1)
    gath_b = (gath_v0, gath_v1)
    sem_b = (sem0, sem1)

    def hash_chunk(c, hidx_v):
        crow0 = c * jnp.int32(CHUNK_ROWS)

        def group_body(g, inner):
            rix = crow0 + g * jnp.int32(LANES) + lanes
            idxv = plsc.load_gather(idx_v, [rix])
            ih = lax.shift_right_logical(idxv, jnp.int32(10))
            il = lax.bitwise_and(idxv, jnp.int32(1023))
            t1v = plsc.load_gather(t1_v, [ih])
            t0v = plsc.load_gather(t0_v, [il])
            base = t1v - pv + t0v
            base = jnp.where(base < 0, base + pv, base)
            t = base - pv + bv
            t = jnp.where(t < 0, t + pv, t)
            pos = (g * jnp.int32(LANES) + lanes) * jnp.int32(EMB)
            for d in range(EMB):
                hh = lax.shift_right_logical(t, jnp.int32(16))
                hl = lax.bitwise_and(t, jnp.int32(0xFFFF))
                t2v = plsc.load_gather(t2_v, [hh])
                r = t2v + hl
                r = jnp.where(r >= wv, r - wv, r)
                plsc.store_scatter(hidx_v, [pos + jnp.int32(d)], r)
                if d != EMB - 1:
                    t = t - pmav
                    t = jnp.where(t < 0, t + pv, t)
            return inner

        lax.fori_loop(jnp.int32(0), jnp.int32(GROUPS), group_body,
                      jnp.int32(0))

    def start_gather(b):
        pltpu.make_async_copy(w_hbm.at[hidx_b[b]], gath_b[b], sem_b[b]).start()

    def drain_chunk(c, b):
        pltpu.make_async_copy(
            w_hbm.at[pl.ds(jnp.int32(0), CHUNK_ELEMS)], gath_b[b],
            sem_b[b]).wait()
        off = (tile_row0 + c * jnp.int32(CHUNK_ROWS)) * jnp.int32(EMB)
        pltpu.sync_copy(gath_b[b], out_hbm.at[pl.ds(off, CHUNK_ELEMS)])

    for b in range(2):
        hash_chunk(jnp.int32(b), hidx_b[b])
        start_gather(b)

    def pipe_body(cc, carry):
        for b in range(2):
            c = cc * jnp.int32(2) + jnp.int32(b)
            drain_chunk(c - jnp.int32(2), b)
            hash_chunk(c, hidx_b[b])
            start_gather(b)
        return carry

    lax.fori_loop(jnp.int32(1), jnp.int32(N_CHUNKS // 2), pipe_body,
                  jnp.int32(0))

    for b in range(2):
        drain_chunk(jnp.int32(N_CHUNKS - 2 + b), b)


def kernel(indices, hashed_weight, random_numbers):
    i_shape = indices.shape
    rn = random_numbers.astype(jnp.int64)
    p, a, b = rn[0], rn[1], rn[2]
    a64 = (a * EMB) % p
    a64k = (a64 * 1024) % p
    j = jnp.arange(1024, dtype=jnp.int64)
    t1 = ((a64k * j) % p).astype(jnp.int32)
    t0 = ((a64 * j) % p).astype(jnp.int32)
    hh = jnp.arange(32768, dtype=jnp.int64)
    t2 = ((hh * 65536) % WEIGHT_SIZE).astype(jnp.int32)
    par = jnp.stack([p, b, p - a, jnp.int64(WEIGHT_SIZE)]).astype(jnp.int32)
    par = jnp.broadcast_to(par[:, None], (4, 16))
    idx_flat = indices.reshape(-1).astype(jnp.int32)
    out = _hash_gather(idx_flat, hashed_weight, t0, t1, t2, par)
    return out.reshape(*i_shape, EMB)


# speedup vs baseline: 317.3792x; 1.3599x over previous
import functools

import jax
import jax.numpy as jnp
from jax import lax
from jax.experimental import pallas as pl
from jax.experimental.pallas import tpu as pltpu
from jax.experimental.pallas import tpu_sc as plsc

EMB = 64
WEIGHT_SIZE = 1000001
N_ROWS = 4096 * 50
NC, NS, LANES = 2, 16, 16
NW = NC * NS
ROWS_PER_TILE = N_ROWS // NW
CHUNK_ROWS = 160
N_CHUNKS = ROWS_PER_TILE // CHUNK_ROWS
CHUNK_ELEMS = CHUNK_ROWS * EMB
GROUPS = CHUNK_ROWS // LANES


_MESH = plsc.VectorSubcoreMesh(core_axis_name="c", subcore_axis_name="s")


@functools.partial(
    pl.kernel,
    mesh=_MESH,
    out_type=jax.ShapeDtypeStruct((N_ROWS * EMB,), jnp.float32),
    scratch_types=[
        pltpu.VMEM((8, 16), jnp.int32),
        pltpu.VMEM((1024,), jnp.int32),
        pltpu.VMEM((1024,), jnp.int32),
        pltpu.VMEM((ROWS_PER_TILE,), jnp.int32),
        pltpu.VMEM((CHUNK_ELEMS,), jnp.int32),
        pltpu.VMEM((CHUNK_ELEMS,), jnp.int32),
        pltpu.VMEM((CHUNK_ELEMS,), jnp.float32),
        pltpu.VMEM((CHUNK_ELEMS,), jnp.float32),
        pltpu.SemaphoreType.DMA,
        pltpu.SemaphoreType.DMA,
        pltpu.VMEM_SHARED((WEIGHT_SIZE,), jnp.float32),
    ],
    compiler_params=pltpu.CompilerParams(needs_layout_passes=False),
)
def _hash_gather(idx_hbm, w_hbm, t0_hbm, t1_hbm, par_hbm, out_hbm,
                 par_v, t0_v, t1_v, idx_v, hidx_v0, hidx_v1,
                 gath_v0, gath_v1, sem0, sem1, w_sh):
    wid = lax.axis_index("s") * jnp.int32(NC) + lax.axis_index("c")
    tile_row0 = wid * jnp.int32(ROWS_PER_TILE)
    pltpu.sync_copy(t0_hbm, t0_v)
    pltpu.sync_copy(t1_hbm, t1_v)
    pltpu.sync_copy(par_hbm, par_v)
    pltpu.sync_copy(idx_hbm.at[pl.ds(tile_row0, ROWS_PER_TILE)], idx_v)

    @pl.when(lax.axis_index("s") == jnp.int32(0))
    def _stage_weights():
        pltpu.sync_copy(w_hbm, w_sh)

    plsc.subcore_barrier()

    pv = par_v[0, :]
    bv = par_v[1, :]
    pmav = par_v[2, :]
    wv = par_v[3, :]
    invwv = plsc.bitcast(par_v[4, :], jnp.float32)
    lanes = lax.iota(jnp.int32, 16)
    hidx_b = (hidx_v0, hidx_v1)
    gath_b = (gath_v0, gath_v1)
    sem_b = (sem0, sem1)

    def hash_chunk(c, hidx_v):
        crow0 = c * jnp.int32(CHUNK_ROWS)

        def group_body(g, inner):
            rix = crow0 + g * jnp.int32(LANES) + lanes
            idxv = plsc.load_gather(idx_v, [rix])
            ih = lax.shift_right_logical(idxv, jnp.int32(10))
            il = lax.bitwise_and(idxv, jnp.int32(1023))
            t1v = plsc.load_gather(t1_v, [ih])
            t0v = plsc.load_gather(t0_v, [il])
            base = t1v - pv + t0v
            base = jnp.where(base < 0, base + pv, base)
            t = base - pv + bv
            t = jnp.where(t < 0, t + pv, t)
            pos = (g * jnp.int32(LANES) + lanes) * jnp.int32(EMB)
            for d in range(EMB):
                qf = t.astype(jnp.float32) * invwv
                qi = qf.astype(jnp.int32)
                r = t - qi * wv
                r = jnp.where(r < 0, r + wv, r)
                r = jnp.where(r >= wv, r - wv, r)
                plsc.store_scatter(hidx_v, [pos + jnp.int32(d)], r)
                if d != EMB - 1:
                    t = t - pmav
                    t = jnp.where(t < 0, t + pv, t)
            return inner

        lax.fori_loop(jnp.int32(0), jnp.int32(GROUPS), group_body,
                      jnp.int32(0))

    def start_gather(b):
        pltpu.make_async_copy(w_sh.at[hidx_b[b]], gath_b[b], sem_b[b]).start()

    def drain_chunk(c, b):
        pltpu.make_async_copy(
            w_hbm.at[pl.ds(jnp.int32(0), CHUNK_ELEMS)], gath_b[b],
            sem_b[b]).wait()
        off = (tile_row0 + c * jnp.int32(CHUNK_ROWS)) * jnp.int32(EMB)
        pltpu.sync_copy(gath_b[b], out_hbm.at[pl.ds(off, CHUNK_ELEMS)])

    for b in range(2):
        hash_chunk(jnp.int32(b), hidx_b[b])
        start_gather(b)

    def pipe_body(cc, carry):
        for b in range(2):
            c = cc * jnp.int32(2) + jnp.int32(b)
            drain_chunk(c - jnp.int32(2), b)
            hash_chunk(c, hidx_b[b])
            start_gather(b)
        return carry

    lax.fori_loop(jnp.int32(1), jnp.int32(N_CHUNKS // 2), pipe_body,
                  jnp.int32(0))

    for b in range(2):
        drain_chunk(jnp.int32(N_CHUNKS - 2 + b), b)


def kernel(indices, hashed_weight, random_numbers):
    i_shape = indices.shape
    rn = random_numbers.astype(jnp.int64)
    p, a, b = rn[0], rn[1], rn[2]
    a64 = (a * EMB) % p
    a64k = (a64 * 1024) % p
    j = jnp.arange(1024, dtype=jnp.int64)
    t1 = ((a64k * j) % p).astype(jnp.int32)
    t0 = ((a64 * j) % p).astype(jnp.int32)
    invw = jnp.float32(1.0) / jnp.float32(WEIGHT_SIZE)
    par = jnp.stack([
        p.astype(jnp.int32), b.astype(jnp.int32), (p - a).astype(jnp.int32),
        jnp.int32(WEIGHT_SIZE), lax.bitcast_convert_type(invw, jnp.int32),
        jnp.int32(0), jnp.int32(0), jnp.int32(0)])
    par = jnp.broadcast_to(par[:, None], (8, 16))
    idx_flat = indices.reshape(-1).astype(jnp.int32)
    out = _hash_gather(idx_flat, hashed_weight, t0, t1, par)
    return out.reshape(*i_shape, EMB)
